# baseline (speedup 1.0000x reference)
import functools

import jax
import jax.numpy as jnp
from jax import lax
from jax.experimental import pallas as pl
from jax.experimental.pallas import tpu as pltpu
from jax.experimental.pallas import tpu_sc as plsc

NC = 2
NS = 16
NW = NC * NS
LANES = 16

CH = 256
N_FULL = 3906
TAIL = 1000000 - N_FULL * CH


def _compact_rows(b64, b128, n_rows):
    for r in range(n_rows):
        for c in range(4):
            v = b64[r, pl.ds(c * LANES, LANES)]
            b128[r // 2, pl.ds((r % 2) * 64 + c * LANES, LANES)] = v


def _compact_body(table_hbm, t2_hbm, b64_a, b64_b, b128_a, b128_b,
                  sr_a, sr_b, sw_a, sw_b):
    wid = lax.axis_index("s") * NC + lax.axis_index("c")
    n_w = (N_FULL - 1 - wid) // NW + 1

    def r0_of(j):
        return pl.multiple_of((wid + NW * j) * CH, CH)

    def start_read(j, b64, sr):
        pltpu.async_copy(table_hbm.at[pl.ds(r0_of(j), CH)], b64, sr)

    def wait_read(j, b64, sr):
        pltpu.make_async_copy(table_hbm.at[pl.ds(r0_of(j), CH)], b64, sr).wait()

    def start_write(j, b128, sw):
        dst0 = pl.multiple_of(r0_of(j) // 2, CH // 2)
        pltpu.async_copy(b128, t2_hbm.at[pl.ds(dst0, CH // 2)], sw)

    def wait_write(b128, sw):
        pltpu.make_async_copy(b128, t2_hbm.at[pl.ds(0, CH // 2)], sw).wait()

    start_read(0, b64_a, sr_a)

    def step(j, carry):
        @pl.when(j % 2 == 0)
        def _():
            @pl.when(j + 1 < n_w)
            def _():
                start_read(j + 1, b64_b, sr_b)
            wait_read(j, b64_a, sr_a)

            @pl.when(j >= 2)
            def _():
                wait_write(b128_a, sw_a)
            _compact_rows(b64_a, b128_a, CH)
            start_write(j, b128_a, sw_a)

        @pl.when(j % 2 == 1)
        def _():
            @pl.when(j + 1 < n_w)
            def _():
                start_read(j + 1, b64_a, sr_a)
            wait_read(j, b64_b, sr_b)

            @pl.when(j >= 2)
            def _():
                wait_write(b128_b, sw_b)
            _compact_rows(b64_b, b128_b, CH)
            start_write(j, b128_b, sw_b)

        return carry

    lax.fori_loop(0, n_w, step, 0)

    @pl.when(n_w >= 2)
    def _():
        wait_write(b128_a, sw_a)
        wait_write(b128_b, sw_b)

    @pl.when(wid == 0)
    def _():
        r0 = N_FULL * CH
        pltpu.sync_copy(table_hbm.at[pl.ds(r0, TAIL)], b64_a.at[pl.ds(0, TAIL)])
        _compact_rows(b64_a, b128_a, TAIL)
        pltpu.sync_copy(b128_a.at[pl.ds(0, TAIL // 2)],
                        t2_hbm.at[pl.ds(r0 // 2, TAIL // 2)])


def _gather_body(bpw, L, table2_hbm, words_hbm, out_hbm,
                 idx_v, gb_a, gb_b, ob_a, ob_b, sg_a, sg_b, sw_a, sw_b):
    wid = lax.axis_index("s") * NC + lax.axis_index("c")
    b0 = pl.multiple_of(wid * bpw, bpw)
    n_pairs = bpw // 2
    NG = (L + LANES - 1) // LANES

    pltpu.sync_copy(words_hbm.at[pl.ds(b0, bpw)], idx_v)

    iota = lax.iota(jnp.int32, LANES)

    def batch_vec(b, g):
        k = g * LANES + iota
        kc = jnp.minimum(k, L - 1)
        bb = jnp.full((LANES,), 0, jnp.int32) + b
        iv = plsc.load_gather(idx_v, [bb, kc])
        return k, kc, iv

    def dma_idx(b, g):
        k, kc, iv = batch_vec(b, g)
        i2 = lax.shift_right_logical(iv, 1)
        return jnp.where(k < L, i2, -1)

    def start_gather(b, gb, sg):
        for g in range(NG):
            pltpu.async_copy(
                table2_hbm.at[plsc.Indices(dma_idx(b, g), ignored_value=-1)],
                gb.at[pl.ds(g * LANES, LANES)], sg)

    def wait_gather(b, gb, sg):
        for g in range(NG):
            pltpu.make_async_copy(
                table2_hbm.at[plsc.Indices(dma_idx(b, g), ignored_value=-1)],
                gb.at[pl.ds(g * LANES, LANES)], sg).wait()

    def extract(b, gb, ob):
        for g in range(NG):
            k, kc, iv = batch_vec(b, g)
            h = (iv & 1) * 64
            for col in range(64):
                cc = jnp.full((LANES,), col, jnp.int32)
                v = plsc.load_gather(gb, [kc, h + cc])
                plsc.store_scatter(ob, [kc, cc], v)

    def start_wb(b, ob, sw):
        pltpu.async_copy(ob, out_hbm.at[b0 + b], sw)

    def wait_wb(ob, sw):
        pltpu.make_async_copy(ob, out_hbm.at[b0], sw).wait()

    start_gather(0, gb_a, sg_a)

    def pair(p, carry):
        e = p * 2
        o = e + 1

        start_gather(o, gb_b, sg_b)
        wait_gather(e, gb_a, sg_a)

        @pl.when(p >= 1)
        def _():
            wait_wb(ob_a, sw_a)
        extract(e, gb_a, ob_a)
        start_wb(e, ob_a, sw_a)

        @pl.when(p + 1 < n_pairs)
        def _():
            start_gather(e + 2, gb_a, sg_a)

        wait_gather(o, gb_b, sg_b)

        @pl.when(p >= 1)
        def _():
            wait_wb(ob_b, sw_b)
        extract(o, gb_b, ob_b)
        start_wb(o, ob_b, sw_b)
        return carry

    lax.fori_loop(0, n_pairs, pair, 0)
    wait_wb(ob_a, sw_a)
    wait_wb(ob_b, sw_b)


@jax.jit
def kernel(words, word_emb):
    B, L = words.shape
    V, D = word_emb.shape
    if words.dtype != jnp.int32:
        words = words.astype(jnp.int32)

    mesh = plsc.VectorSubcoreMesh(core_axis_name="c", subcore_axis_name="s")
    params = pltpu.CompilerParams(needs_layout_passes=False)

    t2 = pl.kernel(
        _compact_body,
        out_type=jax.ShapeDtypeStruct((V // 2, 128), jnp.float32),
        mesh=mesh,
        compiler_params=params,
        scratch_types=[
            pltpu.VMEM((CH, 64), jnp.float32),
            pltpu.VMEM((CH, 64), jnp.float32),
            pltpu.VMEM((CH // 2, 128), jnp.float32),
            pltpu.VMEM((CH // 2, 128), jnp.float32),
            pltpu.SemaphoreType.DMA,
            pltpu.SemaphoreType.DMA,
            pltpu.SemaphoreType.DMA,
            pltpu.SemaphoreType.DMA,
        ],
    )(word_emb)

    bpw = B // NW
    ng = (L + LANES - 1) // LANES
    body = functools.partial(_gather_body, bpw, L)
    out = pl.kernel(
        body,
        out_type=jax.ShapeDtypeStruct((B, L, D), jnp.float32),
        mesh=mesh,
        compiler_params=params,
        scratch_types=[
            pltpu.VMEM((bpw, L), jnp.int32),
            pltpu.VMEM((ng * LANES, 128), jnp.float32),
            pltpu.VMEM((ng * LANES, 128), jnp.float32),
            pltpu.VMEM((L, D), jnp.float32),
            pltpu.VMEM((L, D), jnp.float32),
            pltpu.SemaphoreType.DMA,
            pltpu.SemaphoreType.DMA,
            pltpu.SemaphoreType.DMA,
            pltpu.SemaphoreType.DMA,
        ],
    )(t2, words)
    return out


# speedup vs baseline: 1.0301x; 1.0301x over previous
import functools

import jax
import jax.numpy as jnp
from jax import lax
from jax.experimental import pallas as pl
from jax.experimental.pallas import tpu as pltpu
from jax.experimental.pallas import tpu_sc as plsc

NC = 2
NS = 16
NW = NC * NS
LANES = 16


def _gather_body(bpw, L, table2_hbm, wordsT_hbm, out_hbm,
                 idx_v, idxT, gb_a, gb_b, ob_a, ob_b,
                 sg_a, sg_b, sw_a, sw_b):
    wid = lax.axis_index("s") * NC + lax.axis_index("c")
    b0 = pl.multiple_of(wid * bpw, bpw)
    n_pairs = bpw // 2
    n_idx = bpw * L
    NG = (L + LANES - 1) // LANES

    pltpu.sync_copy(wordsT_hbm.at[:, pl.ds(b0, bpw)], idx_v)

    iota = lax.iota(jnp.int32, LANES)

    def xform(g, carry):
        k = g * LANES + iota
        b = k // L
        l = k - b * L
        iv = plsc.load_gather(idx_v, [l, b])
        plsc.store_scatter(idxT, [b, l], lax.shift_right_logical(iv, 1))
        return carry

    lax.fori_loop(0, n_idx // LANES, xform, 0)

    def start_gather(b, gb, sg):
        pltpu.async_copy(table2_hbm.at[idxT.at[b]], gb, sg)

    def wait_gather(b, gb, sg):
        pltpu.make_async_copy(table2_hbm.at[idxT.at[b]], gb, sg).wait()

    def extract(b, gb, ob):
        bb = jnp.full((LANES,), 0, jnp.int32) + b
        for g in range(NG):
            kc = jnp.minimum(g * LANES + iota, L - 1)
            iv = plsc.load_gather(idx_v, [kc, bb])
            h = (iv & 1) * 64
            for col in range(64):
                cc = jnp.full((LANES,), col, jnp.int32)
                v = plsc.load_gather(gb, [kc, h + cc])
                plsc.store_scatter(ob, [kc, cc], v)

    def start_wb(b, ob, sw):
        pltpu.async_copy(ob, out_hbm.at[b0 + b], sw)

    def wait_wb(ob, sw):
        pltpu.make_async_copy(ob, out_hbm.at[b0], sw).wait()

    start_gather(0, gb_a, sg_a)

    def pair(p, carry):
        e = p * 2
        o = e + 1

        start_gather(o, gb_b, sg_b)
        wait_gather(e, gb_a, sg_a)

        @pl.when(p >= 1)
        def _():
            wait_wb(ob_a, sw_a)
        extract(e, gb_a, ob_a)
        start_wb(e, ob_a, sw_a)

        @pl.when(p + 1 < n_pairs)
        def _():
            start_gather(e + 2, gb_a, sg_a)

        wait_gather(o, gb_b, sg_b)

        @pl.when(p >= 1)
        def _():
            wait_wb(ob_b, sw_b)
        extract(o, gb_b, ob_b)
        start_wb(o, ob_b, sw_b)
        return carry

    lax.fori_loop(0, n_pairs, pair, 0)
    wait_wb(ob_a, sw_a)
    wait_wb(ob_b, sw_b)


@jax.jit
def kernel(words, word_emb):
    B, L = words.shape
    V, D = word_emb.shape
    if words.dtype != jnp.int32:
        words = words.astype(jnp.int32)

    table2 = word_emb.reshape(V // 2, 2 * D)
    wordsT = words.T

    mesh = plsc.VectorSubcoreMesh(core_axis_name="c", subcore_axis_name="s")
    bpw = B // NW
    body = functools.partial(_gather_body, bpw, L)
    out = pl.kernel(
        body,
        out_type=jax.ShapeDtypeStruct((B, L, D), jnp.float32),
        mesh=mesh,
        compiler_params=pltpu.CompilerParams(needs_layout_passes=False),
        scratch_types=[
            pltpu.VMEM((L, bpw), jnp.int32),
            pltpu.VMEM((bpw, L), jnp.int32),
            pltpu.VMEM((L, 2 * D), jnp.float32),
            pltpu.VMEM((L, 2 * D), jnp.float32),
            pltpu.VMEM((L, D), jnp.float32),
            pltpu.VMEM((L, D), jnp.float32),
            pltpu.SemaphoreType.DMA,
            pltpu.SemaphoreType.DMA,
            pltpu.SemaphoreType.DMA,
            pltpu.SemaphoreType.DMA,
        ],
    )(table2, wordsT)
    return out


# speedup vs baseline: 1.6883x; 1.6390x over previous
import functools

import jax
import jax.numpy as jnp
from jax import lax
from jax.experimental import pallas as pl
from jax.experimental.pallas import tpu as pltpu
from jax.experimental.pallas import tpu_sc as plsc

NC = 2
NS = 16
NW = NC * NS
LANES = 16


def _gather_body(bpw, L, table2_hbm, wordsT_hbm, out_hbm,
                 idx_v, idxT, gb_a, gb_b, ob_a, ob_b,
                 sg_a, sg_b, sw_a, sw_b):
    wid = lax.axis_index("s") * NC + lax.axis_index("c")
    b0 = pl.multiple_of(wid * bpw, bpw)
    n_pairs = bpw // 2
    n_idx = bpw * L
    NG = (L + LANES - 1) // LANES

    pltpu.sync_copy(wordsT_hbm.at[:, pl.ds(b0, bpw)], idx_v)

    iota = lax.iota(jnp.int32, LANES)

    @plsc.parallel_loop(0, n_idx // LANES, step=1, unroll=8)
    def _(g):
        k = g * LANES + iota
        b = k // L
        l = k - b * L
        iv = plsc.load_gather(idx_v, [l, b])
        plsc.store_scatter(idxT, [b, l], lax.shift_right_logical(iv, 1))

    def start_gather(b, gb, sg):
        pltpu.async_copy(table2_hbm.at[idxT.at[b]], gb, sg)

    def wait_gather(b, gb, sg):
        pltpu.make_async_copy(table2_hbm.at[idxT.at[b]], gb, sg).wait()

    def extract(b, gb, ob):
        bb = jnp.full((LANES,), 0, jnp.int32) + b

        @plsc.parallel_loop(0, L, step=1, unroll=8)
        def _(r):
            rr = jnp.full((LANES,), 0, jnp.int32) + r
            hv = plsc.load_gather(idx_v, [rr, bb])
            pred = (hv & 1) > 0
            for c in range(4):
                v0 = gb[r, pl.ds(c * LANES, LANES)]
                v1 = gb[r, pl.ds(64 + c * LANES, LANES)]
                ob[r, pl.ds(c * LANES, LANES)] = jnp.where(pred, v1, v0)

    def start_wb(b, ob, sw):
        pltpu.async_copy(ob, out_hbm.at[b0 + b], sw)

    def wait_wb(ob, sw):
        pltpu.make_async_copy(ob, out_hbm.at[b0], sw).wait()

    start_gather(0, gb_a, sg_a)

    def pair(p, carry):
        e = p * 2
        o = e + 1

        start_gather(o, gb_b, sg_b)
        wait_gather(e, gb_a, sg_a)

        @pl.when(p >= 1)
        def _():
            wait_wb(ob_a, sw_a)
        extract(e, gb_a, ob_a)
        start_wb(e, ob_a, sw_a)

        @pl.when(p + 1 < n_pairs)
        def _():
            start_gather(e + 2, gb_a, sg_a)

        wait_gather(o, gb_b, sg_b)

        @pl.when(p >= 1)
        def _():
            wait_wb(ob_b, sw_b)
        extract(o, gb_b, ob_b)
        start_wb(o, ob_b, sw_b)
        return carry

    lax.fori_loop(0, n_pairs, pair, 0)
    wait_wb(ob_a, sw_a)
    wait_wb(ob_b, sw_b)


@jax.jit
def kernel(words, word_emb):
    B, L = words.shape
    V, D = word_emb.shape
    if words.dtype != jnp.int32:
        words = words.astype(jnp.int32)

    table2 = word_emb.reshape(V // 2, 2 * D)
    wordsT = words.T

    mesh = plsc.VectorSubcoreMesh(core_axis_name="c", subcore_axis_name="s")
    bpw = B // NW
    body = functools.partial(_gather_body, bpw, L)
    out = pl.kernel(
        body,
        out_type=jax.ShapeDtypeStruct((B, L, D), jnp.float32),
        mesh=mesh,
        compiler_params=pltpu.CompilerParams(needs_layout_passes=False),
        scratch_types=[
            pltpu.VMEM((L, bpw), jnp.int32),
            pltpu.VMEM((bpw, L), jnp.int32),
            pltpu.VMEM((L, 2 * D), jnp.float32),
            pltpu.VMEM((L, 2 * D), jnp.float32),
            pltpu.VMEM((L, D), jnp.float32),
            pltpu.VMEM((L, D), jnp.float32),
            pltpu.SemaphoreType.DMA,
            pltpu.SemaphoreType.DMA,
            pltpu.SemaphoreType.DMA,
            pltpu.SemaphoreType.DMA,
        ],
    )(table2, wordsT)
    return out
